# baseline (speedup 1.0000x reference)
import jax
import jax.numpy as jnp
from jax.experimental import pallas as pl
from jax.experimental.pallas import tpu as pltpu

S_BLK = 64


def _proj_kernel(tok_ref, feat_ref, w_ref, b_ref, out_ref):
    nb = feat_ref.shape[1]
    for j in range(nb):
        tok_j = tok_ref[j]
        x = feat_ref[:, j, :].astype(jnp.bfloat16)
        w = w_ref[tok_j].astype(jnp.bfloat16)
        acc = jax.lax.dot_general(
            x, w,
            dimension_numbers=(((1,), (1,)), ((), ())),
            preferred_element_type=jnp.float32,
        )
        out_ref[:, j, :] = acc + b_ref[tok_j][None, :]


def kernel(feat, tgt_lang_toks, W, b):
    S, B, C = feat.shape
    E, E_dim, _ = W.shape
    toks = tgt_lang_toks.astype(jnp.int32)

    grid_spec = pltpu.PrefetchScalarGridSpec(
        num_scalar_prefetch=1,
        grid=(S // S_BLK,),
        in_specs=[
            pl.BlockSpec((S_BLK, B, C), lambda s, tok: (s, 0, 0)),
            pl.BlockSpec((E, E_dim, C), lambda s, tok: (0, 0, 0)),
            pl.BlockSpec((E, E_dim), lambda s, tok: (0, 0)),
        ],
        out_specs=pl.BlockSpec((S_BLK, B, E_dim), lambda s, tok: (s, 0, 0)),
    )

    return pl.pallas_call(
        _proj_kernel,
        grid_spec=grid_spec,
        out_shape=jax.ShapeDtypeStruct((S, B, E_dim), feat.dtype),
    )(toks, feat, W, b)


# speedup vs baseline: 1.0975x; 1.0975x over previous
import jax
import jax.numpy as jnp
from jax.experimental import pallas as pl
from jax.experimental.pallas import tpu as pltpu

S_BLK = 512


def _proj_kernel(tok_ref, feat_ref, w_ref, b_ref, out_ref):
    acc = jax.lax.dot_general(
        feat_ref[:], w_ref[0],
        dimension_numbers=(((1,), (1,)), ((), ())),
        preferred_element_type=jnp.float32,
    )
    out_ref[:] = acc + b_ref[0]


def kernel(feat, tgt_lang_toks, W, b):
    S, B, C = feat.shape
    E, E_dim, _ = W.shape
    toks = tgt_lang_toks.astype(jnp.int32)

    feat2 = feat.reshape(S, B * C)
    b3 = b.reshape(E, 1, E_dim)

    grid_spec = pltpu.PrefetchScalarGridSpec(
        num_scalar_prefetch=1,
        grid=(B, S // S_BLK),
        in_specs=[
            pl.BlockSpec((S_BLK, C), lambda j, s, tok: (s, j)),
            pl.BlockSpec((1, E_dim, C), lambda j, s, tok: (tok[j], 0, 0)),
            pl.BlockSpec((1, 1, E_dim), lambda j, s, tok: (tok[j], 0, 0)),
        ],
        out_specs=pl.BlockSpec((S_BLK, E_dim), lambda j, s, tok: (s, j)),
    )

    out2 = pl.pallas_call(
        _proj_kernel,
        grid_spec=grid_spec,
        out_shape=jax.ShapeDtypeStruct((S, B * E_dim), feat.dtype),
    )(toks, feat2, W, b3)
    return out2.reshape(S, B, E_dim)
